# baseline (device time: 48562 ns/iter reference)
import os

import jax
import jax.numpy as jnp
from jax import lax
from jax.experimental import pallas as pl
from jax.experimental.pallas import tpu as pltpu

NCHUNK = 4
A_CHUNK = 256
_NOCOMM = bool(os.environ.get("NOCOMM"))


def kernel(x, assign, W1, W2):
    t, d = x.shape
    n_loc, _, f = W1.shape
    W1 = W1.astype(jnp.bfloat16)
    W2 = W2.astype(jnp.bfloat16)
    assign2d = assign.reshape(t, 1).astype(jnp.int32)
    half = t // 2
    ch = half // NCHUNK

    def body(x_ref, a_ref, w1_ref, w2_ref, out_ref,
             xsb_ref, xr_ref, ar_ref, cs_ref, cr_ref, zs_ref, zr_ref,
             xs_sems, xrcv_sems, as_sem, arcv_sem,
             cs_sems, crcv_sems, zs_sems, zrcv_sems):
        my_x = lax.axis_index("x")
        my_y = lax.axis_index("y")
        xpeer = (1 - my_x, my_y)
        ypeer = (my_x, 1 - my_y)

        if not _NOCOMM:
            bsem = pltpu.get_barrier_semaphore()
            for nbr in (xpeer, ypeer):
                pl.semaphore_signal(bsem, inc=1, device_id=nbr,
                                    device_id_type=pl.DeviceIdType.MESH)
            pl.semaphore_wait(bsem, 2)

        h0 = my_y * half

        rdma_a = pltpu.make_async_remote_copy(
            src_ref=a_ref.at[pl.ds(h0, half), :], dst_ref=ar_ref,
            send_sem=as_sem, recv_sem=arcv_sem,
            device_id=xpeer, device_id_type=pl.DeviceIdType.MESH)
        if not _NOCOMM:
            rdma_a.start()
        xsb_ref[:, :] = x_ref[pl.ds(h0, half), :].astype(jnp.bfloat16)
        rdma_x = []
        for c in range(NCHUNK):
            r = pltpu.make_async_remote_copy(
                src_ref=xsb_ref.at[pl.ds(c * ch, ch), :],
                dst_ref=xr_ref.at[pl.ds(c * ch, ch), :],
                send_sem=xs_sems.at[c], recv_sem=xrcv_sems.at[c],
                device_id=xpeer, device_id_type=pl.DeviceIdType.MESH)
            if not _NOCOMM:
                r.start()
            rdma_x.append(r)

        base = my_x * n_loc

        def moe(x_val, sel):
            x_bf = x_val.astype(jnp.bfloat16)
            acc = jnp.zeros((x_val.shape[0], d), jnp.float32)
            for e in range(n_loc):
                m = (sel == base + e).astype(jnp.float32)
                h = jnp.maximum(
                    jnp.dot(x_bf, w1_ref[e],
                            preferred_element_type=jnp.float32), 0.0)
                y = jnp.dot(h.astype(jnp.bfloat16), w2_ref[e],
                            preferred_element_type=jnp.float32)
                acc = acc + m * y
            return acc

        for c in range(half // A_CHUNK):
            sl = pl.ds(h0 + c * A_CHUNK, A_CHUNK)
            out_ref[sl, :] = moe(x_ref[sl, :], a_ref[sl, :])

        if not _NOCOMM:
            rdma_a.wait()

        rdma_c = []
        for c in range(NCHUNK):
            if not _NOCOMM:
                rdma_x[c].wait()
            sl = pl.ds(c * ch, ch)
            cs_ref[sl, :] = moe(
                xr_ref[sl, :], ar_ref[sl, :]
            ).astype(jnp.bfloat16)
            r = pltpu.make_async_remote_copy(
                src_ref=cs_ref.at[sl, :], dst_ref=cr_ref.at[sl, :],
                send_sem=cs_sems.at[c], recv_sem=crcv_sems.at[c],
                device_id=xpeer, device_id_type=pl.DeviceIdType.MESH)
            if not _NOCOMM:
                r.start()
            rdma_c.append(r)

        rdma_z = []
        for c in range(NCHUNK):
            if not _NOCOMM:
                rdma_c[c].wait()
            slg = pl.ds(h0 + c * ch, ch)
            sl = pl.ds(c * ch, ch)
            z = out_ref[slg, :] + cr_ref[sl, :].astype(jnp.float32)
            out_ref[slg, :] = z
            zs_ref[sl, :] = z.astype(jnp.bfloat16)
            r = pltpu.make_async_remote_copy(
                src_ref=zs_ref.at[sl, :], dst_ref=zr_ref.at[sl, :],
                send_sem=zs_sems.at[c], recv_sem=zrcv_sems.at[c],
                device_id=ypeer, device_id_type=pl.DeviceIdType.MESH)
            if not _NOCOMM:
                r.start()
            rdma_z.append(r)

        oh0 = half - h0
        for c in range(NCHUNK):
            if not _NOCOMM:
                rdma_z[c].wait()
            out_ref[pl.ds(oh0 + c * ch, ch), :] = (
                zr_ref[pl.ds(c * ch, ch), :].astype(jnp.float32))

    return pl.pallas_call(
        body,
        out_shape=jax.ShapeDtypeStruct((t, d), jnp.float32),
        in_specs=[pl.BlockSpec(memory_space=pltpu.VMEM)] * 4,
        out_specs=pl.BlockSpec(memory_space=pltpu.VMEM),
        scratch_shapes=[
            pltpu.VMEM((half, d), jnp.bfloat16),
            pltpu.VMEM((half, d), jnp.bfloat16),
            pltpu.VMEM((half, 1), jnp.int32),
            pltpu.VMEM((half, d), jnp.bfloat16),
            pltpu.VMEM((half, d), jnp.bfloat16),
            pltpu.VMEM((half, d), jnp.bfloat16),
            pltpu.VMEM((half, d), jnp.bfloat16),
            pltpu.SemaphoreType.DMA((NCHUNK,)),
            pltpu.SemaphoreType.DMA((NCHUNK,)),
            pltpu.SemaphoreType.DMA,
            pltpu.SemaphoreType.DMA,
            pltpu.SemaphoreType.DMA((NCHUNK,)),
            pltpu.SemaphoreType.DMA((NCHUNK,)),
            pltpu.SemaphoreType.DMA((NCHUNK,)),
            pltpu.SemaphoreType.DMA((NCHUNK,)),
        ],
        compiler_params=pltpu.CompilerParams(
            collective_id=None if _NOCOMM else 0,
            vmem_limit_bytes=63 * 1024 * 1024),
    )(x, assign2d, W1, W2)
